# baseline (speedup 1.0000x reference)
import functools

import jax
import jax.numpy as jnp
from jax import lax
from jax.experimental import pallas as pl
from jax.experimental.pallas import tpu as pltpu
from jax.experimental.pallas import tpu_sc as plsc

_B = 1024
_T = 50
_D = 32
_DEC = 20
_F = _T * _D

_info = plsc.get_sparse_core_info()
_NC = _info.num_cores
_NS = _info.num_subcores
_NW = _NC * _NS
_NB = _B // _NW


@functools.partial(
    pl.kernel,
    mesh=plsc.VectorSubcoreMesh(core_axis_name="c", subcore_axis_name="s"),
    out_type=jax.ShapeDtypeStruct((_F // 8, _B // 128, 8, 128), jnp.float32),
    compiler_params=pltpu.CompilerParams(
        use_tc_tiling_on_sc=False, needs_layout_passes=False),
    scratch_types=[
        pltpu.VMEM((_NB, _T), jnp.int32),
        pltpu.VMEM((_NB * _T, _D), jnp.float32),
        pltpu.VMEM((_F // 8, 8, _NB), jnp.float32),
        pltpu.SemaphoreType.DMA,
    ],
)
def _sc_gather(idx_hbm, table_hbm, out_hbm, idx_v, rows_v, t_v, gsem):
    wid = lax.axis_index("s") * _NC + lax.axis_index("c")
    base = wid * _NB

    pltpu.sync_copy(idx_hbm.at[pl.ds(base, _NB)], idx_v)

    gathers = [
        pltpu.async_copy(table_hbm.at[idx_v.at[i]],
                         rows_v.at[pl.ds(i * _T, _T)], gsem)
        for i in range(_NB)
    ]
    for g in gathers:
        g.wait()

    iota = lax.iota(jnp.int32, 16)
    tr_off = lax.shift_right_logical(iota, 3)
    r_idx = iota & 7

    def row_body(i, carry):
        col = jnp.full((16,), i, jnp.int32)
        for g in range(_F // 16):
            f0 = g * 16
            vals = rows_v[i * _T + g // 2, pl.ds((g % 2) * 16, 16)]
            tr = tr_off + (f0 // 8)
            plsc.store_scatter(t_v, [tr, r_idx, col], vals)
        return carry

    lax.fori_loop(0, _NB, row_body, 0)

    pltpu.sync_copy(
        t_v,
        out_hbm.at[:, wid // 4, :, pl.ds((wid % 4) * _NB, _NB)])


@functools.partial(
    pl.kernel,
    mesh=plsc.VectorSubcoreMesh(core_axis_name="c", subcore_axis_name="s"),
    out_type=jax.ShapeDtypeStruct((_DEC, _F, _B), jnp.float32),
    scratch_types=[
        pltpu.VMEM((56, _B), jnp.float32),
        pltpu.SemaphoreType.DMA,
    ],
)
def _sc_bcast(gt_hbm, out_hbm, chunk_v, wsem):
    wid = lax.axis_index("s") * _NC + lax.axis_index("c")

    is_big = wid < 8
    off_big = wid * 56
    off_small = 8 * 56 + (wid - 8) * 48
    f0 = jnp.where(is_big, off_big, off_small)
    f0 = pl.multiple_of(f0, 8)

    @pl.when(is_big)
    def _():
        pltpu.sync_copy(gt_hbm.at[pl.ds(f0, 56)], chunk_v)
        writes = [
            pltpu.async_copy(chunk_v, out_hbm.at[j, pl.ds(f0, 56)], wsem)
            for j in range(_DEC)
        ]
        for w in writes:
            w.wait()

    @pl.when(jnp.logical_not(is_big))
    def _():
        pltpu.sync_copy(gt_hbm.at[pl.ds(f0, 48)], chunk_v.at[pl.ds(0, 48)])
        writes = [
            pltpu.async_copy(chunk_v.at[pl.ds(0, 48)],
                             out_hbm.at[j, pl.ds(f0, 48)], wsem)
            for j in range(_DEC)
        ]
        for w in writes:
            w.wait()


def kernel(inputs, decoder_length, table):
    del decoder_length
    t5 = _sc_gather(inputs, table)
    gt = t5.transpose(0, 2, 1, 3).reshape(_F, _B)
    out = _sc_bcast(gt)
    return out.transpose(2, 0, 1)


# speedup vs baseline: 1.1447x; 1.1447x over previous
import functools

import jax
import jax.numpy as jnp
from jax import lax
from jax.experimental import pallas as pl
from jax.experimental.pallas import tpu as pltpu
from jax.experimental.pallas import tpu_sc as plsc

_B = 1024
_T = 50
_D = 32
_DEC = 20
_F = _T * _D

_info = plsc.get_sparse_core_info()
_NC = _info.num_cores
_NS = _info.num_subcores
_NW = _NC * _NS
_NB = _B // _NW


@functools.partial(
    pl.kernel,
    mesh=plsc.VectorSubcoreMesh(core_axis_name="c", subcore_axis_name="s"),
    out_type=jax.ShapeDtypeStruct((_F // 8, _B // 128, 8, 128), jnp.float32),
    compiler_params=pltpu.CompilerParams(
        use_tc_tiling_on_sc=False, needs_layout_passes=False),
    scratch_types=[
        pltpu.VMEM((_NB, _T), jnp.int32),
        pltpu.VMEM((_NB * _T, _D), jnp.float32),
        pltpu.VMEM((_F // 8, 8, _NB + 1), jnp.float32),
        pltpu.SemaphoreType.DMA,
    ],
)
def _sc_gather(idx_hbm, table_hbm, out_hbm, idx_v, rows_v, t_v, gsem):
    wid = lax.axis_index("s") * _NC + lax.axis_index("c")
    base = wid * _NB

    pltpu.sync_copy(idx_hbm.at[pl.ds(base, _NB)], idx_v)

    gathers = [
        pltpu.async_copy(table_hbm.at[idx_v.at[i]],
                         rows_v.at[pl.ds(i * _T, _T)], gsem)
        for i in range(_NB)
    ]
    for g in gathers:
        g.wait()

    iota = lax.iota(jnp.int32, 16)
    tr_off = lax.shift_right_logical(iota, 3)
    r_idx = iota & 7

    def row_body(i, carry):
        col = jnp.full((16,), i, jnp.int32)
        for g in range(_F // 16):
            f0 = g * 16
            vals = rows_v[i * _T + g // 2, pl.ds((g % 2) * 16, 16)]
            tr = tr_off + (f0 // 8)
            plsc.store_scatter(t_v, [tr, r_idx, col], vals)
        return carry

    lax.fori_loop(0, _NB, row_body, 0)

    pltpu.sync_copy(
        t_v.at[:, :, pl.ds(0, _NB)],
        out_hbm.at[:, wid // 4, :, pl.ds((wid % 4) * _NB, _NB)])


@functools.partial(
    pl.kernel,
    mesh=plsc.VectorSubcoreMesh(core_axis_name="c", subcore_axis_name="s"),
    out_type=jax.ShapeDtypeStruct((_DEC, _F, _B), jnp.float32),
    scratch_types=[
        pltpu.VMEM((56, _B), jnp.float32),
        pltpu.SemaphoreType.DMA,
    ],
)
def _sc_bcast(gt_hbm, out_hbm, chunk_v, wsem):
    wid = lax.axis_index("s") * _NC + lax.axis_index("c")

    is_big = wid < 8
    off_big = wid * 56
    off_small = 8 * 56 + (wid - 8) * 48
    f0 = jnp.where(is_big, off_big, off_small)
    f0 = pl.multiple_of(f0, 8)

    @pl.when(is_big)
    def _():
        pltpu.sync_copy(gt_hbm.at[pl.ds(f0, 56)], chunk_v)
        writes = [
            pltpu.async_copy(chunk_v, out_hbm.at[j, pl.ds(f0, 56)], wsem)
            for j in range(_DEC)
        ]
        for w in writes:
            w.wait()

    @pl.when(jnp.logical_not(is_big))
    def _():
        pltpu.sync_copy(gt_hbm.at[pl.ds(f0, 48)], chunk_v.at[pl.ds(0, 48)])
        writes = [
            pltpu.async_copy(chunk_v.at[pl.ds(0, 48)],
                             out_hbm.at[j, pl.ds(f0, 48)], wsem)
            for j in range(_DEC)
        ]
        for w in writes:
            w.wait()


def kernel(inputs, decoder_length, table):
    del decoder_length
    t5 = _sc_gather(inputs, table)
    gt = t5.transpose(0, 2, 1, 3).reshape(_F, _B)
    out = _sc_bcast(gt)
    return out.transpose(2, 0, 1)


# speedup vs baseline: 1.2519x; 1.0937x over previous
import functools

import jax
import jax.numpy as jnp
from jax import lax
from jax.experimental import pallas as pl
from jax.experimental.pallas import tpu as pltpu
from jax.experimental.pallas import tpu_sc as plsc

_B = 1024
_T = 50
_D = 32
_DEC = 20
_F = _T * _D

_info = plsc.get_sparse_core_info()
_NC = _info.num_cores
_NS = _info.num_subcores
_NW = _NC * _NS
_NB = _B // _NW


@functools.partial(
    pl.kernel,
    mesh=plsc.VectorSubcoreMesh(core_axis_name="c", subcore_axis_name="s"),
    out_type=jax.ShapeDtypeStruct((_DEC, _F // 8, _B // 128, 8, 128),
                                  jnp.float32),
    compiler_params=pltpu.CompilerParams(
        use_tc_tiling_on_sc=False, needs_layout_passes=False),
    scratch_types=[
        pltpu.VMEM((_NB, _T), jnp.int32),
        pltpu.VMEM((_NB * _T, _D), jnp.float32),
        pltpu.VMEM((_F // 8, 8, _NB + 1), jnp.float32),
        pltpu.SemaphoreType.DMA,
        pltpu.SemaphoreType.DMA,
    ],
)
def _sc_gather(idx_hbm, table_hbm, out_hbm, idx_v, rows_v, t_v, gsem, wsem):
    wid = lax.axis_index("s") * _NC + lax.axis_index("c")
    base = wid * _NB

    pltpu.sync_copy(idx_hbm.at[pl.ds(base, _NB)], idx_v)

    gathers = [
        pltpu.async_copy(table_hbm.at[idx_v.at[i]],
                         rows_v.at[pl.ds(i * _T, _T)], gsem)
        for i in range(_NB)
    ]
    for g in gathers:
        g.wait()

    iota = lax.iota(jnp.int32, 16)
    tr_off = lax.shift_right_logical(iota, 3)
    r_idx = iota & 7

    def row_body(i, carry):
        col = jnp.full((16,), i, jnp.int32)
        for g in range(_F // 16):
            f0 = g * 16
            vals = rows_v[i * _T + g // 2, pl.ds((g % 2) * 16, 16)]
            tr = tr_off + (f0 // 8)
            plsc.store_scatter(t_v, [tr, r_idx, col], vals)
        return carry

    lax.fori_loop(0, _NB, row_body, 0)

    writes = [
        pltpu.async_copy(
            t_v.at[:, :, pl.ds(0, _NB)],
            out_hbm.at[j, :, wid // 4, :, pl.ds((wid % 4) * _NB, _NB)],
            wsem)
        for j in range(_DEC)
    ]
    for w in writes:
        w.wait()


def kernel(inputs, decoder_length, table):
    del decoder_length
    out6 = _sc_gather(inputs, table)
    return (out6.transpose(2, 4, 0, 1, 3)
            .reshape(_B, _DEC, _F))
